# baseline (speedup 1.0000x reference)
import jax
import jax.numpy as jnp
from jax.experimental import pallas as pl
from jax.experimental.pallas import tpu as pltpu

N = 10000
D = 128
BR = 10000
NBLK = N // BR
BR2 = 10000
NBLK2 = N // BR2
BC = 1000
NCH = N // BC
TAU = 0.5
EPS = 1e-12
LOG2E_OVER_TAU = 1.4426950408889634 / TAU


def _fused_kernel(x_ref, w1_ref, b1_ref, g_ref, bb_ref, w2_ref, b2_ref,
                  wp_ref, bp_ref, out_ref,
                  s_scr, cs_scr, scale_scr, shift_scr, abf_scr, bbf_scr,
                  acc_scr):
    t = pl.program_id(0)

    @pl.when(t == 0)
    def _init():
        s_scr[...] = jnp.zeros_like(s_scr)
        cs_scr[...] = jnp.zeros_like(cs_scr)
        acc_scr[0, 0] = 0.0

    @pl.when(t < NBLK)
    def _stats():
        x = x_ref[...]
        s_scr[...] += jax.lax.dot_general(
            x, x, (((0,), (0,)), ((), ())), preferred_element_type=jnp.float32)
        cs_scr[...] += jnp.sum(x, axis=0, keepdims=True)

    @pl.when(t == NBLK - 1)
    def _bn_stats():
        w1 = w1_ref[...]
        mu = cs_scr[...] * (1.0 / N)
        mc = jax.lax.dot_general(
            mu, w1, (((1,), (1,)), ((), ())), preferred_element_type=jnp.float32)
        mh = mc + b1_ref[...]
        t1 = jax.lax.dot_general(
            w1, s_scr[...], (((1,), (0,)), ((), ())),
            preferred_element_type=jnp.float32)
        p = jax.lax.dot_general(
            t1, w1, (((1,), (1,)), ((), ())), preferred_element_type=jnp.float32)
        rows = jax.lax.broadcasted_iota(jnp.int32, p.shape, 0)
        cols = jax.lax.broadcasted_iota(jnp.int32, p.shape, 1)
        q = jnp.sum(jnp.where(rows == cols, p, 0.0), axis=0, keepdims=True)
        var = q * (1.0 / N) - mc * mc
        sc = g_ref[...] * jax.lax.rsqrt(var + 1e-5)
        scale_scr[...] = sc
        shift_scr[...] = bb_ref[...] - mh * sc

    @pl.when((t >= NBLK) & (t < 2 * NBLK))
    def _embed():
        j = t - NBLK
        x = x_ref[...]
        h = jax.lax.dot_general(
            x, w1_ref[...], (((1,), (1,)), ((), ())),
            preferred_element_type=jnp.float32) + b1_ref[...]
        h = jnp.maximum(h * scale_scr[...] + shift_scr[...], 0.0)
        zf = jax.lax.dot_general(
            h, w2_ref[...], (((1,), (1,)), ((), ())),
            preferred_element_type=jnp.float32) + b2_ref[...]
        nz = jnp.sqrt(jnp.sum(zf * zf, axis=1, keepdims=True))
        zh = zf / jnp.maximum(nz, EPS)
        proj = jax.lax.dot_general(
            zf, wp_ref[...], (((1,), (1,)), ((), ())),
            preferred_element_type=jnp.float32) + bp_ref[...]
        np_ = jnp.sqrt(jnp.sum(proj * proj, axis=1, keepdims=True))
        zu = proj / jnp.maximum(np_, EPS)
        bbf_scr[pl.ds(j * BR, BR), :] = zh.astype(jnp.bfloat16)
        abf_scr[pl.ds(j * BR, BR), :] = (zu * LOG2E_OVER_TAU).astype(jnp.bfloat16)
        acc_scr[0, 0] += jnp.sum(zu * zh) * (1.0 / TAU)

    @pl.when(t >= 2 * NBLK)
    def _loss():
        j = t - 2 * NBLK
        abf = abf_scr[pl.ds(j * BR2, BR2), :]
        rs = jnp.zeros((BR2, 1), jnp.float32)
        for c in range(NCH):
            sim2 = jax.lax.dot_general(
                abf, bbf_scr[pl.ds(c * BC, BC), :], (((1,), (1,)), ((), ())),
                preferred_element_type=jnp.float32)
            rs += jnp.sum(jnp.exp2(sim2), axis=1, keepdims=True)
        acc_scr[0, 0] += -jnp.sum(jnp.log(rs + EPS))

    @pl.when(t == 2 * NBLK + NBLK2 - 1)
    def _fin():
        out_ref[0, 0] = -acc_scr[0, 0] * (1.0 / N)


def kernel(graph, feat, W1, b1, bn_g, bn_b, W2, b2, Wp, bp):
    del graph
    b1r = b1.reshape(1, D)
    b2r = b2.reshape(1, D)
    bpr = bp.reshape(1, D)
    gr = bn_g.reshape(1, D)
    bbr = bn_b.reshape(1, D)

    def _feat_map(t):
        return (jnp.where(t < 2 * NBLK, jax.lax.rem(t, NBLK), 0), 0)

    loss = pl.pallas_call(
        _fused_kernel,
        grid=(2 * NBLK + NBLK2,),
        in_specs=[
            pl.BlockSpec((BR, D), _feat_map),
            pl.BlockSpec((D, D), lambda t: (0, 0)),
            pl.BlockSpec((1, D), lambda t: (0, 0)),
            pl.BlockSpec((1, D), lambda t: (0, 0)),
            pl.BlockSpec((1, D), lambda t: (0, 0)),
            pl.BlockSpec((D, D), lambda t: (0, 0)),
            pl.BlockSpec((1, D), lambda t: (0, 0)),
            pl.BlockSpec((D, D), lambda t: (0, 0)),
            pl.BlockSpec((1, D), lambda t: (0, 0)),
        ],
        out_specs=pl.BlockSpec(memory_space=pltpu.SMEM),
        out_shape=jax.ShapeDtypeStruct((1, 1), jnp.float32),
        scratch_shapes=[
            pltpu.VMEM((D, D), jnp.float32),
            pltpu.VMEM((1, D), jnp.float32),
            pltpu.VMEM((1, D), jnp.float32),
            pltpu.VMEM((1, D), jnp.float32),
            pltpu.VMEM((N, D), jnp.bfloat16),
            pltpu.VMEM((N, D), jnp.bfloat16),
            pltpu.SMEM((1, 1), jnp.float32),
        ],
        compiler_params=pltpu.CompilerParams(
            vmem_limit_bytes=100 * 1024 * 1024),
    )(feat, W1, b1r, gr, bbr, W2, b2r, Wp, bpr)

    return jnp.reshape(loss, ())


# speedup vs baseline: 1.2799x; 1.2799x over previous
import jax
import jax.numpy as jnp
from jax.experimental import pallas as pl
from jax.experimental.pallas import tpu as pltpu

N = 10000
D = 128
BR = 5000
NBLK = N // BR
BR2 = 5000
NBLK2 = N // BR2
BC = 1000
NCH = N // BC
TAU = 0.5
EPS = 1e-12
LOG2E_OVER_TAU = 1.4426950408889634 / TAU


def _fused_kernel(x_ref, w1_ref, b1_ref, g_ref, bb_ref, w2_ref, b2_ref,
                  wp_ref, bp_ref, out_ref,
                  s_scr, cs_scr, scale_scr, shift_scr, abf_scr, bbf_scr,
                  acc_scr):
    t = pl.program_id(0)

    @pl.when(t == 0)
    def _init():
        s_scr[...] = jnp.zeros_like(s_scr)
        cs_scr[...] = jnp.zeros_like(cs_scr)
        acc_scr[0, 0] = 0.0

    @pl.when(t < NBLK)
    def _stats():
        x = x_ref[...]
        s_scr[...] += jax.lax.dot_general(
            x, x, (((0,), (0,)), ((), ())), preferred_element_type=jnp.float32)
        cs_scr[...] += jnp.sum(x, axis=0, keepdims=True)

    @pl.when(t == NBLK - 1)
    def _bn_stats():
        w1 = w1_ref[...]
        mu = cs_scr[...] * (1.0 / N)
        mc = jax.lax.dot_general(
            mu, w1, (((1,), (1,)), ((), ())), preferred_element_type=jnp.float32)
        mh = mc + b1_ref[...]
        t1 = jax.lax.dot_general(
            w1, s_scr[...], (((1,), (0,)), ((), ())),
            preferred_element_type=jnp.float32)
        p = jax.lax.dot_general(
            t1, w1, (((1,), (1,)), ((), ())), preferred_element_type=jnp.float32)
        rows = jax.lax.broadcasted_iota(jnp.int32, p.shape, 0)
        cols = jax.lax.broadcasted_iota(jnp.int32, p.shape, 1)
        q = jnp.sum(jnp.where(rows == cols, p, 0.0), axis=0, keepdims=True)
        var = q * (1.0 / N) - mc * mc
        sc = g_ref[...] * jax.lax.rsqrt(var + 1e-5)
        scale_scr[...] = sc
        shift_scr[...] = bb_ref[...] - mh * sc

    @pl.when((t >= NBLK) & (t < 2 * NBLK))
    def _embed():
        j = t - NBLK
        x = x_ref[...]
        h = jax.lax.dot_general(
            x, w1_ref[...], (((1,), (1,)), ((), ())),
            preferred_element_type=jnp.float32) + b1_ref[...]
        h = jnp.maximum(h * scale_scr[...] + shift_scr[...], 0.0)
        zf = jax.lax.dot_general(
            h, w2_ref[...], (((1,), (1,)), ((), ())),
            preferred_element_type=jnp.float32) + b2_ref[...]
        nz = jnp.sqrt(jnp.sum(zf * zf, axis=1, keepdims=True))
        zh = zf / jnp.maximum(nz, EPS)
        proj = jax.lax.dot_general(
            zf, wp_ref[...], (((1,), (1,)), ((), ())),
            preferred_element_type=jnp.float32) + bp_ref[...]
        np_ = jnp.sqrt(jnp.sum(proj * proj, axis=1, keepdims=True))
        zu = proj / jnp.maximum(np_, EPS)
        bbf_scr[pl.ds(j * BR, BR), :] = zh.astype(jnp.bfloat16)
        abf_scr[pl.ds(j * BR, BR), :] = (zu * LOG2E_OVER_TAU).astype(jnp.bfloat16)
        acc_scr[0, 0] += jnp.sum(zu * zh) * (1.0 / TAU)

    @pl.when(t >= 2 * NBLK)
    def _loss():
        j = t - 2 * NBLK
        abf = abf_scr[pl.ds(j * BR2, BR2), :]
        rs = jnp.zeros((BR2, 1), jnp.float32)
        for c in range(NCH):
            sim2 = jax.lax.dot_general(
                abf, bbf_scr[pl.ds(c * BC, BC), :], (((1,), (1,)), ((), ())),
                preferred_element_type=jnp.float32)
            rs += jnp.sum(jnp.exp2(sim2), axis=1, keepdims=True)
        acc_scr[0, 0] += -jnp.sum(jnp.log(rs + EPS))

    @pl.when(t == 2 * NBLK + NBLK2 - 1)
    def _fin():
        out_ref[0, 0] = -acc_scr[0, 0] * (1.0 / N)


def kernel(graph, feat, W1, b1, bn_g, bn_b, W2, b2, Wp, bp):
    del graph
    b1r = b1.reshape(1, D)
    b2r = b2.reshape(1, D)
    bpr = bp.reshape(1, D)
    gr = bn_g.reshape(1, D)
    bbr = bn_b.reshape(1, D)

    def _feat_map(t):
        return (jnp.where(t < 2 * NBLK, jax.lax.rem(t, NBLK), 0), 0)

    loss = pl.pallas_call(
        _fused_kernel,
        grid=(2 * NBLK + NBLK2,),
        in_specs=[
            pl.BlockSpec((BR, D), _feat_map),
            pl.BlockSpec((D, D), lambda t: (0, 0)),
            pl.BlockSpec((1, D), lambda t: (0, 0)),
            pl.BlockSpec((1, D), lambda t: (0, 0)),
            pl.BlockSpec((1, D), lambda t: (0, 0)),
            pl.BlockSpec((D, D), lambda t: (0, 0)),
            pl.BlockSpec((1, D), lambda t: (0, 0)),
            pl.BlockSpec((D, D), lambda t: (0, 0)),
            pl.BlockSpec((1, D), lambda t: (0, 0)),
        ],
        out_specs=pl.BlockSpec(memory_space=pltpu.SMEM),
        out_shape=jax.ShapeDtypeStruct((1, 1), jnp.float32),
        scratch_shapes=[
            pltpu.VMEM((D, D), jnp.float32),
            pltpu.VMEM((1, D), jnp.float32),
            pltpu.VMEM((1, D), jnp.float32),
            pltpu.VMEM((1, D), jnp.float32),
            pltpu.VMEM((N, D), jnp.bfloat16),
            pltpu.VMEM((N, D), jnp.bfloat16),
            pltpu.SMEM((1, 1), jnp.float32),
        ],
        compiler_params=pltpu.CompilerParams(
            vmem_limit_bytes=100 * 1024 * 1024),
    )(feat, W1, b1r, gr, bbr, W2, b2r, Wp, bpr)

    return jnp.reshape(loss, ())
